# baseline (speedup 1.0000x reference)
import functools

import jax
import jax.numpy as jnp
from jax import lax
from jax.experimental import pallas as pl
from jax.experimental.pallas import tpu as pltpu
from jax.experimental.pallas import tpu_sc as plsc


_NC, _NS = 2, 16
_CHUNK = 16
_NBUF = 3


def kernel(x, pos_embed):
    b, s, d = x.shape
    nw = _NC * _NS
    rows = s // nw
    n = _CHUNK
    mesh = plsc.VectorSubcoreMesh(core_axis_name="c", subcore_axis_name="s")

    @functools.partial(
        pl.kernel,
        mesh=mesh,
        out_type=jax.ShapeDtypeStruct((b, s, 2 * d), x.dtype),
        scratch_types=(
            [pltpu.VMEM((n, 2 * d), jnp.float32)] * _NBUF
            + [pltpu.SemaphoreType.DMA] * (2 * _NBUF)
        ),
    )
    def k(x_hbm, pe_hbm, out_hbm, *scratch):
        bufs = scratch[:_NBUF]
        sin = scratch[_NBUF : 2 * _NBUF]
        sout = scratch[2 * _NBUF :]
        wid = lax.axis_index("s") * _NC + lax.axis_index("c")
        s0 = wid * rows

        work = []
        for bi in range(b):
            for c in range(rows // n):
                work.append((bi, c * n))

        t = len(work)
        gat = [[] for _ in range(_NBUF)]
        pend = [None] * _NBUF

        def issue_gathers(i):
            slot = i % _NBUF
            if pend[slot] is not None:
                pend[slot].wait()
                pend[slot] = None
            bi, off = work[i]
            gat[slot] = [
                pltpu.async_copy(
                    x_hbm.at[bi, pl.ds(s0 + off, n), :],
                    bufs[slot].at[:, pl.ds(0, d)],
                    sin[slot],
                ),
                pltpu.async_copy(
                    pe_hbm.at[pl.ds(s0 + off, n), :],
                    bufs[slot].at[:, pl.ds(d, d)],
                    sin[slot],
                ),
            ]

        issue_gathers(0)
        issue_gathers(1)
        for i in range(t):
            slot = i % _NBUF
            for h in gat[slot]:
                h.wait()
            bi, off = work[i]
            pend[slot] = pltpu.async_copy(
                bufs[slot], out_hbm.at[bi, pl.ds(s0 + off, n), :], sout[slot]
            )
            if i + 2 < t:
                issue_gathers(i + 2)
        for slot in range(_NBUF):
            if pend[slot] is not None:
                pend[slot].wait()

    return k(x, pos_embed)


# speedup vs baseline: 1.2125x; 1.2125x over previous
import functools

import jax
import jax.numpy as jnp
from jax import lax
from jax.experimental import pallas as pl
from jax.experimental.pallas import tpu as pltpu
from jax.experimental.pallas import tpu_sc as plsc


_NC, _NS = 2, 16
_CHUNK = 32
_NBUF = 3
_BS = 512


def _sc_pe_broadcast(pos_embed, b, s, d, dtype):
    nw = _NC * _NS
    rows = s // nw
    n = _CHUNK
    mesh = plsc.VectorSubcoreMesh(core_axis_name="c", subcore_axis_name="s")

    @functools.partial(
        pl.kernel,
        mesh=mesh,
        out_type=jax.ShapeDtypeStruct((b, s, 2 * d), dtype),
        scratch_types=(
            [pltpu.VMEM((n, d), jnp.float32)] * _NBUF
            + [pltpu.SemaphoreType.DMA] * (2 * _NBUF)
        ),
    )
    def k(pe_hbm, out_hbm, *scratch):
        bufs = scratch[:_NBUF]
        sin = scratch[_NBUF : 2 * _NBUF]
        sout = scratch[2 * _NBUF :]
        wid = lax.axis_index("s") * _NC + lax.axis_index("c")
        s0 = wid * rows

        t = rows // n
        gat = [None] * _NBUF
        pend = [[] for _ in range(_NBUF)]

        def issue_gather(i):
            slot = i % _NBUF
            for h in pend[slot]:
                h.wait()
            pend[slot] = []
            gat[slot] = pltpu.async_copy(
                pe_hbm.at[pl.ds(s0 + i * n, n), :], bufs[slot], sin[slot]
            )

        issue_gather(0)
        if t > 1:
            issue_gather(1)
        for i in range(t):
            slot = i % _NBUF
            gat[slot].wait()
            pend[slot] = [
                pltpu.async_copy(
                    bufs[slot],
                    out_hbm.at[bi, pl.ds(s0 + i * n, n), pl.ds(d, d)],
                    sout[slot],
                )
                for bi in range(b)
            ]
            if i + 2 < t:
                issue_gather(i + 2)
        for slot in range(_NBUF):
            for h in pend[slot]:
                h.wait()

    return k(pos_embed)


def _tc_body(x_ref, out_pe_ref, out_ref):
    del out_pe_ref
    out_ref[...] = x_ref[...]


def kernel(x, pos_embed):
    b, s, d = x.shape
    out_pe = _sc_pe_broadcast(pos_embed, b, s, d, x.dtype)
    return pl.pallas_call(
        _tc_body,
        grid=(s // _BS, b),
        in_specs=[
            pl.BlockSpec((1, _BS, d), lambda i, j: (j, i, 0)),
            pl.BlockSpec(memory_space=pl.ANY),
        ],
        out_specs=pl.BlockSpec((1, _BS, d), lambda i, j: (j, i, 0)),
        out_shape=jax.ShapeDtypeStruct((b, s, 2 * d), x.dtype),
        input_output_aliases={1: 0},
    )(x, out_pe)


# speedup vs baseline: 1.2674x; 1.0453x over previous
import functools

import jax
import jax.numpy as jnp
from jax import lax
from jax.experimental import pallas as pl
from jax.experimental.pallas import tpu as pltpu
from jax.experimental.pallas import tpu_sc as plsc


_NC, _NS = 2, 16
_CHUNK = 32
_NBUF = 3
_BS = 1024


def _sc_pe_broadcast(pos_embed, b, s, d, dtype):
    nw = _NC * _NS
    rows = s // nw
    n = _CHUNK
    mesh = plsc.VectorSubcoreMesh(core_axis_name="c", subcore_axis_name="s")

    @functools.partial(
        pl.kernel,
        mesh=mesh,
        out_type=jax.ShapeDtypeStruct((b, s, 2 * d), dtype),
        scratch_types=(
            [pltpu.VMEM((n, d), jnp.float32)] * _NBUF
            + [pltpu.SemaphoreType.DMA] * (2 * _NBUF)
        ),
    )
    def k(pe_hbm, out_hbm, *scratch):
        bufs = scratch[:_NBUF]
        sin = scratch[_NBUF : 2 * _NBUF]
        sout = scratch[2 * _NBUF :]
        wid = lax.axis_index("s") * _NC + lax.axis_index("c")
        s0 = wid * rows

        t = rows // n
        gat = [None] * _NBUF
        pend = [[] for _ in range(_NBUF)]

        def issue_gather(i):
            slot = i % _NBUF
            for h in pend[slot]:
                h.wait()
            pend[slot] = []
            gat[slot] = pltpu.async_copy(
                pe_hbm.at[pl.ds(s0 + i * n, n), :], bufs[slot], sin[slot]
            )

        issue_gather(0)
        if t > 1:
            issue_gather(1)
        for i in range(t):
            slot = i % _NBUF
            gat[slot].wait()
            pend[slot] = [
                pltpu.async_copy(
                    bufs[slot],
                    out_hbm.at[bi, pl.ds(s0 + i * n, n), pl.ds(d, d)],
                    sout[slot],
                )
                for bi in range(b)
            ]
            if i + 2 < t:
                issue_gather(i + 2)
        for slot in range(_NBUF):
            for h in pend[slot]:
                h.wait()

    return k(pos_embed)


def _tc_body(x_ref, out_pe_ref, out_ref):
    del out_pe_ref
    out_ref[...] = x_ref[...]


def kernel(x, pos_embed):
    b, s, d = x.shape
    out_pe = _sc_pe_broadcast(pos_embed, b, s, d, x.dtype)
    return pl.pallas_call(
        _tc_body,
        grid=(s // _BS, b),
        in_specs=[
            pl.BlockSpec((1, _BS, d), lambda i, j: (j, i, 0)),
            pl.BlockSpec(memory_space=pl.ANY),
        ],
        out_specs=pl.BlockSpec((1, _BS, d), lambda i, j: (j, i, 0)),
        out_shape=jax.ShapeDtypeStruct((b, s, 2 * d), x.dtype),
        input_output_aliases={1: 0},
    )(x, out_pe)


# speedup vs baseline: 1.2807x; 1.0105x over previous
import functools

import jax
import jax.numpy as jnp
from jax import lax
from jax.experimental import pallas as pl
from jax.experimental.pallas import tpu as pltpu
from jax.experimental.pallas import tpu_sc as plsc


_NC, _NS = 2, 16
_CHUNK = 32
_NBUF = 3
_BS = 2048


def _sc_pe_broadcast(pos_embed, b, s, d, dtype):
    nw = _NC * _NS
    rows = s // nw
    n = _CHUNK
    mesh = plsc.VectorSubcoreMesh(core_axis_name="c", subcore_axis_name="s")

    @functools.partial(
        pl.kernel,
        mesh=mesh,
        out_type=jax.ShapeDtypeStruct((b, s, 2 * d), dtype),
        scratch_types=(
            [pltpu.VMEM((n, d), jnp.float32)] * _NBUF
            + [pltpu.SemaphoreType.DMA] * (2 * _NBUF)
        ),
    )
    def k(pe_hbm, out_hbm, *scratch):
        bufs = scratch[:_NBUF]
        sin = scratch[_NBUF : 2 * _NBUF]
        sout = scratch[2 * _NBUF :]
        wid = lax.axis_index("s") * _NC + lax.axis_index("c")
        s0 = wid * rows

        t = rows // n
        gat = [None] * _NBUF
        pend = [[] for _ in range(_NBUF)]

        def issue_gather(i):
            slot = i % _NBUF
            for h in pend[slot]:
                h.wait()
            pend[slot] = []
            gat[slot] = pltpu.async_copy(
                pe_hbm.at[pl.ds(s0 + i * n, n), :], bufs[slot], sin[slot]
            )

        issue_gather(0)
        if t > 1:
            issue_gather(1)
        for i in range(t):
            slot = i % _NBUF
            gat[slot].wait()
            pend[slot] = [
                pltpu.async_copy(
                    bufs[slot],
                    out_hbm.at[bi, pl.ds(s0 + i * n, n), pl.ds(d, d)],
                    sout[slot],
                )
                for bi in range(b)
            ]
            if i + 2 < t:
                issue_gather(i + 2)
        for slot in range(_NBUF):
            for h in pend[slot]:
                h.wait()

    return k(pos_embed)


def _tc_body(x_ref, out_pe_ref, out_ref):
    del out_pe_ref
    out_ref[...] = x_ref[...]


def kernel(x, pos_embed):
    b, s, d = x.shape
    out_pe = _sc_pe_broadcast(pos_embed, b, s, d, x.dtype)
    return pl.pallas_call(
        _tc_body,
        grid=(s // _BS, b),
        in_specs=[
            pl.BlockSpec((1, _BS, d), lambda i, j: (j, i, 0)),
            pl.BlockSpec(memory_space=pl.ANY),
        ],
        out_specs=pl.BlockSpec((1, _BS, d), lambda i, j: (j, i, 0)),
        out_shape=jax.ShapeDtypeStruct((b, s, 2 * d), x.dtype),
        input_output_aliases={1: 0},
    )(x, out_pe)
